# baseline (speedup 1.0000x reference)
import functools

import numpy as np
import jax
import jax.numpy as jnp
from jax import lax
from jax.experimental import pallas as pl
from jax.experimental.pallas import tpu as pltpu
from jax.experimental.pallas import tpu_sc as plsc

EPS = 1e-6

_blades = [0b000, 0b001, 0b010, 0b100, 0b011, 0b101, 0b110, 0b111]
_grade = np.array([bin(b).count("1") for b in _blades])
_idx_of = {b: i for i, b in enumerate(_blades)}


def _blade_product(a, b):
    swaps = 0
    aa = a >> 1
    while aa:
        swaps += bin(aa & b).count("1")
        aa >>= 1
    return a ^ b, (-1.0 if swaps % 2 else 1.0)


_CAYLEY_ENTRIES = []
for _i, _bi in enumerate(_blades):
    for _k, _bk in enumerate(_blades):
        _bo, _s = _blade_product(_bi, _bk)
        _CAYLEY_ENTRIES.append((_i, _idx_of[_bo], _k, _s))

_CAYLEY = np.zeros((8, 8, 8), dtype=np.float32)
for (_i, _j, _k, _s) in _CAYLEY_ENTRIES:
    _CAYLEY[_i, _j, _k] = _s
_GP_PATHS = np.zeros((4, 4, 4), dtype=bool)
for _gi in range(4):
    for _gj in range(4):
        for _gk in range(4):
            _m = ((_grade[:, None, None] == _gi) & (_grade[None, :, None] == _gj)
                  & (_grade[None, None, :] == _gk))
            _GP_PATHS[_gi, _gj, _gk] = bool(np.any(_CAYLEY[_m] != 0))
_GP_IDX = np.nonzero(_GP_PATHS)

_GRADE_OF_BLADE = [0, 1, 1, 1, 2, 2, 2, 3]


def _prep_layer(p, pieces):
    W = p["W"]
    sgp = p["sgp"]
    Co = W.shape[0]
    w4 = jnp.zeros((Co, 4, 4, 4), W.dtype).at[
        :, _GP_IDX[0], _GP_IDX[1], _GP_IDX[2]].set(sgp["gp_w"])
    coef = jnp.stack([w4[:, _grade[i], _grade[j], _grade[k]] * s
                      for (i, j, k, s) in _CAYLEY_ENTRIES])

    def blockdiag(Wsub):
        Co2, pc = Wsub.shape[0], Wsub.shape[1]
        M = jnp.zeros((8 * Co2, 8 * pc), W.dtype)
        for j in range(8):
            M = M.at[j * Co2:(j + 1) * Co2, j * pc:(j + 1) * pc].set(
                Wsub[:, :, _grade[j]])
        return M

    mats, c0 = [], 0
    for pc in pieces:
        mats.append(blockdiag(W[:, c0:c0 + pc, :]))
        c0 += pc
    return {
        "Wms": tuple(mats),
        "b": p["b"][:, None],
        "sa": jnp.transpose(p["silu_a"][0][:, _grade])[..., None],
        "sb": jnp.transpose(p["silu_b"][0][:, _grade])[..., None],
        "Wlm": blockdiag(sgp["Wl"]),
        "bl": sgp["bl"][:, None],
        "Wrm": blockdiag(sgp["Wr"]),
        "na": jnp.transpose(sgp["norm_a"][0][:, _grade])[..., None],
        "coef": coef[..., None],
        "lna": p["ln_a"][0][:, None],
    }


_LKEYS = ["b", "sa", "sb", "Wlm", "bl", "Wrm", "na", "coef", "lna"]


def _flatten_layers(layers):
    out = []
    for L in layers:
        out.extend(L["Wms"])
        out.extend(L[k] for k in _LKEYS)
    return out


def _read_layers(refs, npieces):
    layers, pos = [], 0
    for np_ in npieces:
        L = {"Wms": tuple(refs[pos + i][...] for i in range(np_))}
        pos += np_
        for k in _LKEYS:
            L[k] = refs[pos][...]
            pos += 1
        layers.append(L)
    return layers


def _mvlin(xs, Wms, b=None):
    acc = None
    for xp, Wm in zip(xs, Wms):
        X = xp.reshape(xp.shape[0] * xp.shape[1], xp.shape[2])
        t = jnp.dot(Wm, X, preferred_element_type=jnp.float32)
        acc = t if acc is None else acc + t
    Co = Wms[0].shape[0] // 8
    acc = acc.reshape(8, Co, acc.shape[1])
    if b is not None:
        acc = jnp.concatenate([acc[0:1] + b[None], acc[1:]], axis=0)
    return acc


def _qs(x):
    sq = x * x
    return [sq[0], sq[1] + sq[2] + sq[3], sq[4] + sq[5] + sq[6], sq[7]]


def _rep(vals):
    return jnp.stack([vals[_GRADE_OF_BLADE[j]] for j in range(8)], axis=0)


def _mv_silu(x, sa, sb):
    q = _qs(x)
    inv = jnp.stack([x[0], q[1], q[1], q[1], q[2], q[2], q[2], q[3]], axis=0)
    gate = jax.nn.sigmoid(sa * inv + sb)
    return gate * x


def _grade_norm(x, na):
    q = _qs(x)
    norms = _rep([jnp.sqrt(v + EPS) for v in q])
    nn = jax.nn.sigmoid(na) * (norms - 1.0) + 1.0
    return x / (nn + EPS)


def _gp(x, xr, coef):
    ys = [None] * 8
    for e, (i, j, k, s) in enumerate(_CAYLEY_ENTRIES):
        t = coef[e] * (x[i] * xr[k])
        ys[j] = t if ys[j] is None else ys[j] + t
    return jnp.stack(ys, axis=0)


def _mv_layernorm(x, lna):
    q = _qs(x)
    tot = q[0] + q[1] + q[2] + q[3]
    norms = jnp.sqrt(tot + EPS)
    nm = norms.mean(axis=0, keepdims=True) + EPS
    return (lna[None] * x) / nm[None]


def _cemlp(xs, layers):
    for L in layers:
        x = _mvlin(xs, L["Wms"], L["b"])
        x = _mv_silu(x, L["sa"], L["sb"])
        xr = _grade_norm(_mvlin([x], (L["Wrm"],)), L["na"])
        left = _mvlin([x], (L["Wlm"],), L["bl"])
        x = (left + _gp(x, xr, L["coef"])) * np.float32(1.0 / np.sqrt(2.0))
        x = _mv_layernorm(x, L["lna"])
        xs = [x]
    return xs[0]


def _phie_call(srows, rrows, attr, wflat, T):
    E = srows.shape[0]
    nt = E // T

    def body(*refs):
        s_ref, r_ref, a_ref = refs[0:3]
        o_ref = refs[-1]
        layers = _read_layers(refs[3:-1], (2, 1))
        d = s_ref[...] - r_ref[...]
        xh = d.T.reshape(8, 8, T)
        xa = a_ref[...].T.reshape(8, 4, T)
        y = _cemlp([xh, xa], layers)
        o_ref[...] = y.reshape(64, T).T

    full = lambda w: pl.BlockSpec(w.shape, lambda i: (0,) * w.ndim)
    return pl.pallas_call(
        body,
        grid=(nt,),
        in_specs=[pl.BlockSpec((T, 64), lambda i: (i, 0)),
                  pl.BlockSpec((T, 64), lambda i: (i, 0)),
                  pl.BlockSpec((T, 32), lambda i: (i, 0))]
                 + [full(w) for w in wflat],
        out_specs=pl.BlockSpec((T, 64), lambda i: (i, 0)),
        out_shape=jax.ShapeDtypeStruct((E, 64), jnp.float32),
    )(srows, rrows, attr, *wflat)


def _phih_call(hperm, sums, cnts, wflat, T):
    N = hperm.shape[0]
    nt = N // T

    def body(*refs):
        h_ref, m_ref, c_ref = refs[0:3]
        o_ref = refs[-1]
        layers = _read_layers(refs[3:-1], (2, 1))
        hT = h_ref[...].T.reshape(8, 8, T)
        cT = c_ref[...].T
        inv = 1.0 / jnp.maximum(cT[0:1], 1.0)
        mT = m_ref[...].T.reshape(8, 8, T) * inv[None]
        y = _cemlp([hT, mT], layers) + hT
        o_ref[...] = y.reshape(64, T).T

    full = lambda w: pl.BlockSpec(w.shape, lambda i: (0,) * w.ndim)
    return pl.pallas_call(
        body,
        grid=(nt,),
        in_specs=[pl.BlockSpec((T, 64), lambda i: (i, 0)),
                  pl.BlockSpec((T, 64), lambda i: (i, 0)),
                  pl.BlockSpec((T, 16), lambda i: (i, 0))]
                 + [full(w) for w in wflat],
        out_specs=pl.BlockSpec((T, 64), lambda i: (i, 0)),
        out_shape=jax.ShapeDtypeStruct((N, 64), jnp.float32),
    )(hperm, sums, cnts, *wflat)


_SC_CH = 128


def _sc_gather(hperm, s2, r2):
    NROW = s2.shape[0]
    E = NROW * _SC_CH
    NW = 32
    NB = 2
    NIT = NROW // (NW * NB)
    assert NIT % 2 == 0 and NIT >= 4
    BE = NB * _SC_CH
    mesh = plsc.VectorSubcoreMesh(core_axis_name="c", subcore_axis_name="s",
                                  num_cores=2, num_subcores=16)

    @functools.partial(
        pl.kernel,
        out_type=(jax.ShapeDtypeStruct((E, 64), jnp.float32),
                  jax.ShapeDtypeStruct((E, 64), jnp.float32)),
        mesh=mesh,
        compiler_params=pltpu.CompilerParams(use_tc_tiling_on_sc=False),
        scratch_types=[
            pltpu.VMEM((2, NB, _SC_CH), jnp.int32),
            pltpu.VMEM((2, NB, _SC_CH), jnp.int32),
            pltpu.VMEM((2, BE, 64), jnp.float32),
            pltpu.VMEM((2, BE, 64), jnp.float32),
            pltpu.SemaphoreType.DMA((2,)),
            pltpu.SemaphoreType.DMA((2,)),
            pltpu.SemaphoreType.DMA((2,)),
        ],
    )
    def k(hp, sh, rh, so, ro, sidx, ridx, srows, rrows, semi, semg, semw):
        wid = lax.axis_index("s") * 2 + lax.axis_index("c")
        row_base = wid * NIT * NB

        def idx_load(i, p):
            rr = row_base + i * NB
            pltpu.async_copy(sh.at[pl.ds(rr, NB)], sidx.at[p], semi.at[p])
            pltpu.async_copy(rh.at[pl.ds(rr, NB)], ridx.at[p], semi.at[p])

        def idx_wait(i, p):
            rr = row_base + i * NB
            pltpu.make_async_copy(sh.at[pl.ds(rr, NB)], sidx.at[p],
                                  semi.at[p]).wait()
            pltpu.make_async_copy(rh.at[pl.ds(rr, NB)], ridx.at[p],
                                  semi.at[p]).wait()

        def g_fire(p):
            for r in range(NB):
                pltpu.async_copy(hp.at[sidx.at[p, r]],
                                 srows.at[p, pl.ds(r * _SC_CH, _SC_CH)],
                                 semg.at[p])
                pltpu.async_copy(hp.at[ridx.at[p, r]],
                                 rrows.at[p, pl.ds(r * _SC_CH, _SC_CH)],
                                 semg.at[p])

        def g_wait(p):
            for r in range(NB):
                pltpu.make_async_copy(hp.at[sidx.at[p, r]],
                                      srows.at[p, pl.ds(r * _SC_CH, _SC_CH)],
                                      semg.at[p]).wait()
                pltpu.make_async_copy(hp.at[ridx.at[p, r]],
                                      rrows.at[p, pl.ds(r * _SC_CH, _SC_CH)],
                                      semg.at[p]).wait()

        def w_fire(i, p):
            off = (row_base + i * NB) * _SC_CH
            pltpu.async_copy(srows.at[p], so.at[pl.ds(off, BE)], semw.at[p])
            pltpu.async_copy(rrows.at[p], ro.at[pl.ds(off, BE)], semw.at[p])

        def w_drain(i, p):
            off = (row_base + i * NB) * _SC_CH
            pltpu.make_async_copy(srows.at[p], so.at[pl.ds(off, BE)],
                                  semw.at[p]).wait()
            pltpu.make_async_copy(rrows.at[p], ro.at[pl.ds(off, BE)],
                                  semw.at[p]).wait()

        idx_load(0, 0)
        idx_wait(0, 0)
        g_fire(0)
        idx_load(1, 1)

        def step(i, p):
            g_wait(p)
            w_fire(i, p)

            @pl.when(i + 1 < NIT)
            def _():
                idx_wait(i + 1, 1 - p)

                @pl.when(i + 2 < NIT)
                def _():
                    idx_load(i + 2, p)

                @pl.when(i >= 1)
                def _():
                    w_drain(i - 1, 1 - p)

                g_fire(1 - p)

        def body(io, carry):
            step(io * 2, 0)
            step(io * 2 + 1, 1)
            return carry

        lax.fori_loop(0, NIT // 2, body, 0)
        w_drain(NIT - 2, 0)
        w_drain(NIT - 1, 1)

    return k(hperm, s2, r2)


def _sc_scatter(m, s2, n_nodes):
    NROW = s2.shape[0]
    NHALF = n_nodes // 2
    NS = 16
    NHP = ((NHALF + 1 + NS - 1) // NS) * NS
    NIT = NROW // NS
    assert NIT % 2 == 0
    RPT = NHP // NS
    zeros64 = jnp.zeros((_SC_CH, 64), jnp.float32)
    mesh = plsc.VectorSubcoreMesh(core_axis_name="c", subcore_axis_name="s",
                                  num_cores=2, num_subcores=16)

    @functools.partial(
        pl.kernel,
        out_type=jax.ShapeDtypeStruct((2 * NHP, 64), jnp.float32),
        mesh=mesh,
        compiler_params=pltpu.CompilerParams(use_tc_tiling_on_sc=False),
        scratch_types=[
            pltpu.VMEM((2, _SC_CH), jnp.int32),
            pltpu.VMEM((2, _SC_CH), jnp.int32),
            pltpu.VMEM((2, _SC_CH, 64), jnp.float32),
            pltpu.VMEM_SHARED((NHP, 64), jnp.float32),
            pltpu.SemaphoreType.DMA((2,)),
            pltpu.SemaphoreType.DMA((2,)),
        ],
    )
    def k(m_h, s_h, z64_h, sums_o, sidx, idxb, mbuf, sums_sh, seml, semsc):
        c = lax.axis_index("c")
        s = lax.axis_index("s")
        base = c * NHALF
        row0 = s * RPT

        pltpu.sync_copy(z64_h, mbuf.at[0])
        o = 0
        while o < RPT:
            n = min(_SC_CH, RPT - o)
            pltpu.sync_copy(mbuf.at[0, pl.ds(0, n)],
                            sums_sh.at[pl.ds(row0 + o, n)])
            o += n
        plsc.subcore_barrier()

        def m_load(i, p):
            r0 = s * NIT + i
            pltpu.async_copy(s_h.at[r0], sidx.at[p], seml.at[p])
            pltpu.async_copy(m_h.at[pl.ds(r0 * _SC_CH, _SC_CH)], mbuf.at[p],
                             seml.at[p])

        def m_wait(i, p):
            r0 = s * NIT + i
            pltpu.make_async_copy(s_h.at[r0], sidx.at[p], seml.at[p]).wait()
            pltpu.make_async_copy(m_h.at[pl.ds(r0 * _SC_CH, _SC_CH)],
                                  mbuf.at[p], seml.at[p]).wait()

        def sc_drain(p):
            pltpu.make_async_copy(mbuf.at[p], sums_sh.at[idxb.at[p]],
                                  semsc.at[p]).wait()

        m_load(0, 0)

        def step(i, p):
            m_wait(i, p)
            for v in range(_SC_CH // 16):
                sl = sidx[p, pl.ds(v * 16, 16)]
                loc = sl - base
                ok = (loc >= 0) & (loc < NHALF)
                idxb[p, pl.ds(v * 16, 16)] = jnp.where(ok, loc, NHALF)
            pltpu.async_copy(mbuf.at[p], sums_sh.at[idxb.at[p]],
                             semsc.at[p], add=True)

            @pl.when(i >= 1)
            def _():
                sc_drain(1 - p)

            @pl.when(i + 1 < NIT)
            def _():
                m_load(i + 1, 1 - p)

        def body(io, carry):
            step(io * 2, 0)
            step(io * 2 + 1, 1)
            return carry

        lax.fori_loop(0, NIT // 2, body, 0)
        sc_drain(1)
        plsc.subcore_barrier()

        out0 = c * NHP + row0
        o = 0
        while o < RPT:
            n = min(_SC_CH, RPT - o)
            pltpu.sync_copy(sums_sh.at[pl.ds(row0 + o, n)],
                            mbuf.at[0, pl.ds(0, n)])
            pltpu.sync_copy(mbuf.at[0, pl.ds(0, n)],
                            sums_o.at[pl.ds(out0 + o, n)])
            o += n

    sums2 = k(m, s2, zeros64)
    return jnp.concatenate([sums2[:NHALF], sums2[NHP:NHP + NHALF]], axis=0)


def _sc_counts(s2, n_nodes):
    NROW = s2.shape[0]
    NHALF = n_nodes // 2
    NS = 16
    NHP = ((NHALF + 1 + NS - 1) // NS) * NS
    NB = 2
    NIT = NROW // (NS * NB)
    assert NIT % 2 == 0
    RPT = NHP // NS
    zeros16 = jnp.zeros((_SC_CH, 16), jnp.float32)
    ones16 = jnp.ones((_SC_CH, 16), jnp.float32)
    mesh = plsc.VectorSubcoreMesh(core_axis_name="c", subcore_axis_name="s",
                                  num_cores=2, num_subcores=16)

    @functools.partial(
        pl.kernel,
        out_type=jax.ShapeDtypeStruct((2 * NHP, 16), jnp.float32),
        mesh=mesh,
        compiler_params=pltpu.CompilerParams(use_tc_tiling_on_sc=False),
        scratch_types=[
            pltpu.VMEM((2, NB, _SC_CH), jnp.int32),
            pltpu.VMEM((2, NB, _SC_CH), jnp.int32),
            pltpu.VMEM((_SC_CH, 16), jnp.float32),
            pltpu.VMEM_SHARED((NHP, 16), jnp.float32),
            pltpu.SemaphoreType.DMA((2,)),
            pltpu.SemaphoreType.DMA((2,)),
        ],
    )
    def k(s_h, z16_h, one_h, cnts_o, sidx, idxb, obuf, cnts_sh, seml, semsc):
        c = lax.axis_index("c")
        s = lax.axis_index("s")
        base = c * NHALF
        row0 = s * RPT

        pltpu.sync_copy(z16_h, obuf)
        o = 0
        while o < RPT:
            n = min(_SC_CH, RPT - o)
            pltpu.sync_copy(obuf.at[pl.ds(0, n)],
                            cnts_sh.at[pl.ds(row0 + o, n)])
            o += n
        pltpu.sync_copy(one_h, obuf)
        plsc.subcore_barrier()

        def q_load(i, p):
            r0 = (s * NIT + i) * NB
            pltpu.async_copy(s_h.at[pl.ds(r0, NB)], sidx.at[p], seml.at[p])

        def q_wait(i, p):
            r0 = (s * NIT + i) * NB
            pltpu.make_async_copy(s_h.at[pl.ds(r0, NB)], sidx.at[p],
                                  seml.at[p]).wait()

        def q_drain(p):
            for r in range(NB):
                pltpu.make_async_copy(obuf, cnts_sh.at[idxb.at[p, r]],
                                      semsc.at[p]).wait()

        q_load(0, 0)

        def step(i, p):
            q_wait(i, p)
            for r in range(NB):
                for v in range(_SC_CH // 16):
                    sl = sidx[p, r, pl.ds(v * 16, 16)]
                    loc = sl - base
                    ok = (loc >= 0) & (loc < NHALF)
                    idxb[p, r, pl.ds(v * 16, 16)] = jnp.where(ok, loc, NHALF)
            for r in range(NB):
                pltpu.async_copy(obuf, cnts_sh.at[idxb.at[p, r]],
                                 semsc.at[p], add=True)

            @pl.when(i >= 1)
            def _():
                q_drain(1 - p)

            @pl.when(i + 1 < NIT)
            def _():
                q_load(i + 1, 1 - p)

        def body(io, carry):
            step(io * 2, 0)
            step(io * 2 + 1, 1)
            return carry

        lax.fori_loop(0, NIT // 2, body, 0)
        q_drain(1)
        plsc.subcore_barrier()

        out0 = c * NHP + row0
        o = 0
        while o < RPT:
            n = min(_SC_CH, RPT - o)
            pltpu.sync_copy(cnts_sh.at[pl.ds(row0 + o, n)],
                            obuf.at[pl.ds(0, n)])
            pltpu.sync_copy(obuf.at[pl.ds(0, n)],
                            cnts_o.at[pl.ds(out0 + o, n)])
            o += n

    cnts2 = k(s2, zeros16, ones16)
    return jnp.concatenate([cnts2[:NHALF], cnts2[NHP:NHP + NHALF]], axis=0)


def kernel(h, edge_index, edge_attr, params):
    N, F = h.shape[0], h.shape[1]
    E = edge_index.shape[1]
    EC = edge_attr.shape[1]
    assert F == 8 and EC == 4 and N % 2 == 0

    phie = [_prep_layer(p, pc) for p, pc in zip(params["phi_e"], [(8, 4), (8,)])]
    phih = [_prep_layer(p, pc) for p, pc in zip(params["phi_h"], [(8, 8), (8,)])]
    we = _flatten_layers(phie)
    wh = _flatten_layers(phih)

    hperm = jnp.transpose(h, (0, 2, 1)).reshape(N, 64)
    aperm = jnp.transpose(edge_attr, (0, 2, 1)).reshape(E, 32)

    EPAD = ((E + 32 * _SC_CH - 1) // (32 * _SC_CH)) * (32 * _SC_CH)
    senders = edge_index[0]
    NR = EPAD // _SC_CH
    s2 = jnp.pad(senders, (0, EPAD - E)).reshape(NR, _SC_CH)
    r2 = jnp.pad(edge_index[1], (0, EPAD - E)).reshape(NR, _SC_CH)
    s2s = jnp.pad(senders, (0, EPAD - E),
                  constant_values=N).reshape(NR, _SC_CH)
    aperm = jnp.pad(aperm, ((0, EPAD - E), (0, 0)))

    srows, rrows = _sc_gather(hperm, s2, r2)

    cnts = _sc_counts(s2s, N)

    m_e = _phie_call(srows, rrows, aperm, we, T=4096)

    sums = _sc_scatter(m_e, s2s, N)

    TN = 1024
    NPAD = ((N + TN - 1) // TN) * TN
    hperm_p = jnp.pad(hperm, ((0, NPAD - N), (0, 0)))
    sums_p = jnp.pad(sums, ((0, NPAD - N), (0, 0)))
    cnts_p = jnp.pad(cnts, ((0, NPAD - N), (0, 0)))
    y = _phih_call(hperm_p, sums_p, cnts_p, wh, T=TN)

    return jnp.transpose(y[:N].reshape(N, 8, 8), (0, 2, 1))


# speedup vs baseline: 1.0129x; 1.0129x over previous
import functools

import numpy as np
import jax
import jax.numpy as jnp
from jax import lax
from jax.experimental import pallas as pl
from jax.experimental.pallas import tpu as pltpu
from jax.experimental.pallas import tpu_sc as plsc

EPS = 1e-6

_blades = [0b000, 0b001, 0b010, 0b100, 0b011, 0b101, 0b110, 0b111]
_grade = np.array([bin(b).count("1") for b in _blades])
_idx_of = {b: i for i, b in enumerate(_blades)}


def _blade_product(a, b):
    swaps = 0
    aa = a >> 1
    while aa:
        swaps += bin(aa & b).count("1")
        aa >>= 1
    return a ^ b, (-1.0 if swaps % 2 else 1.0)


_CAYLEY_ENTRIES = []
for _i, _bi in enumerate(_blades):
    for _k, _bk in enumerate(_blades):
        _bo, _s = _blade_product(_bi, _bk)
        _CAYLEY_ENTRIES.append((_i, _idx_of[_bo], _k, _s))

_CAYLEY = np.zeros((8, 8, 8), dtype=np.float32)
for (_i, _j, _k, _s) in _CAYLEY_ENTRIES:
    _CAYLEY[_i, _j, _k] = _s
_GP_PATHS = np.zeros((4, 4, 4), dtype=bool)
for _gi in range(4):
    for _gj in range(4):
        for _gk in range(4):
            _m = ((_grade[:, None, None] == _gi) & (_grade[None, :, None] == _gj)
                  & (_grade[None, None, :] == _gk))
            _GP_PATHS[_gi, _gj, _gk] = bool(np.any(_CAYLEY[_m] != 0))
_GP_IDX = np.nonzero(_GP_PATHS)

_GRADE_OF_BLADE = [0, 1, 1, 1, 2, 2, 2, 3]


def _prep_layer(p, pieces):
    W = p["W"]
    sgp = p["sgp"]
    Co = W.shape[0]
    w4 = jnp.zeros((Co, 4, 4, 4), W.dtype).at[
        :, _GP_IDX[0], _GP_IDX[1], _GP_IDX[2]].set(sgp["gp_w"])
    coef = jnp.stack([w4[:, _grade[i], _grade[j], _grade[k]] * s
                      for (i, j, k, s) in _CAYLEY_ENTRIES])

    def blockdiag(Wsub):
        Co2, pc = Wsub.shape[0], Wsub.shape[1]
        M = jnp.zeros((8 * Co2, 8 * pc), W.dtype)
        for j in range(8):
            M = M.at[j * Co2:(j + 1) * Co2, j * pc:(j + 1) * pc].set(
                Wsub[:, :, _grade[j]])
        return M

    mats, c0 = [], 0
    for pc in pieces:
        mats.append(blockdiag(W[:, c0:c0 + pc, :]))
        c0 += pc
    return {
        "Wms": tuple(mats),
        "b": p["b"][:, None],
        "sa": jnp.transpose(p["silu_a"][0][:, _grade])[..., None],
        "sb": jnp.transpose(p["silu_b"][0][:, _grade])[..., None],
        "Wlm": blockdiag(sgp["Wl"]),
        "bl": sgp["bl"][:, None],
        "Wrm": blockdiag(sgp["Wr"]),
        "na": jnp.transpose(sgp["norm_a"][0][:, _grade])[..., None],
        "coef": coef[..., None],
        "lna": p["ln_a"][0][:, None],
    }


_LKEYS = ["b", "sa", "sb", "Wlm", "bl", "Wrm", "na", "coef", "lna"]


def _flatten_layers(layers):
    out = []
    for L in layers:
        out.extend(L["Wms"])
        out.extend(L[k] for k in _LKEYS)
    return out


def _read_layers(refs, npieces):
    layers, pos = [], 0
    for np_ in npieces:
        L = {"Wms": tuple(refs[pos + i][...] for i in range(np_))}
        pos += np_
        for k in _LKEYS:
            L[k] = refs[pos][...]
            pos += 1
        layers.append(L)
    return layers


def _mvlin(xs, Wms, b=None):
    acc = None
    for xp, Wm in zip(xs, Wms):
        X = xp.reshape(xp.shape[0] * xp.shape[1], xp.shape[2])
        t = jnp.dot(Wm, X, preferred_element_type=jnp.float32)
        acc = t if acc is None else acc + t
    Co = Wms[0].shape[0] // 8
    acc = acc.reshape(8, Co, acc.shape[1])
    if b is not None:
        acc = jnp.concatenate([acc[0:1] + b[None], acc[1:]], axis=0)
    return acc


def _qs(x):
    sq = x * x
    return [sq[0], sq[1] + sq[2] + sq[3], sq[4] + sq[5] + sq[6], sq[7]]


def _rep(vals):
    return jnp.stack([vals[_GRADE_OF_BLADE[j]] for j in range(8)], axis=0)


def _mv_silu(x, sa, sb):
    q = _qs(x)
    inv = jnp.stack([x[0], q[1], q[1], q[1], q[2], q[2], q[2], q[3]], axis=0)
    gate = jax.nn.sigmoid(sa * inv + sb)
    return gate * x


def _grade_norm(x, na):
    q = _qs(x)
    norms = _rep([jnp.sqrt(v + EPS) for v in q])
    nn = jax.nn.sigmoid(na) * (norms - 1.0) + 1.0
    return x / (nn + EPS)


def _gp(x, xr, coef):
    ys = [None] * 8
    for e, (i, j, k, s) in enumerate(_CAYLEY_ENTRIES):
        t = coef[e] * (x[i] * xr[k])
        ys[j] = t if ys[j] is None else ys[j] + t
    return jnp.stack(ys, axis=0)


def _mv_layernorm(x, lna):
    q = _qs(x)
    tot = q[0] + q[1] + q[2] + q[3]
    norms = jnp.sqrt(tot + EPS)
    nm = norms.mean(axis=0, keepdims=True) + EPS
    return (lna[None] * x) / nm[None]


def _cemlp(xs, layers):
    for L in layers:
        x = _mvlin(xs, L["Wms"], L["b"])
        x = _mv_silu(x, L["sa"], L["sb"])
        xr = _grade_norm(_mvlin([x], (L["Wrm"],)), L["na"])
        left = _mvlin([x], (L["Wlm"],), L["bl"])
        x = (left + _gp(x, xr, L["coef"])) * np.float32(1.0 / np.sqrt(2.0))
        x = _mv_layernorm(x, L["lna"])
        xs = [x]
    return xs[0]


def _phie_call(srows, rrows, attr, wflat, T):
    E = srows.shape[0]
    nt = E // T

    def body(*refs):
        s_ref, r_ref, a_ref = refs[0:3]
        o_ref = refs[-1]
        layers = _read_layers(refs[3:-1], (2, 1))
        d = s_ref[...] - r_ref[...]
        xh = d.T.reshape(8, 8, T)
        xa = a_ref[...].T.reshape(8, 4, T)
        y = _cemlp([xh, xa], layers)
        o_ref[...] = y.reshape(64, T).T

    full = lambda w: pl.BlockSpec(w.shape, lambda i: (0,) * w.ndim)
    return pl.pallas_call(
        body,
        grid=(nt,),
        in_specs=[pl.BlockSpec((T, 64), lambda i: (i, 0)),
                  pl.BlockSpec((T, 64), lambda i: (i, 0)),
                  pl.BlockSpec((T, 32), lambda i: (i, 0))]
                 + [full(w) for w in wflat],
        out_specs=pl.BlockSpec((T, 64), lambda i: (i, 0)),
        out_shape=jax.ShapeDtypeStruct((E, 64), jnp.float32),
    )(srows, rrows, attr, *wflat)


def _phih_call(hperm, sums, c0, c1, wflat, T):
    N = hperm.shape[0]
    nt = N // T

    def body(*refs):
        h_ref, m_ref, c0_ref, c1_ref = refs[0:4]
        o_ref = refs[-1]
        layers = _read_layers(refs[4:-1], (2, 1))
        hT = h_ref[...].T.reshape(8, 8, T)
        cT = c0_ref[...].T + c1_ref[...].T
        inv = 1.0 / jnp.maximum(cT[0:1], 1.0)
        mT = m_ref[...].T.reshape(8, 8, T) * inv[None]
        y = _cemlp([hT, mT], layers) + hT
        o_ref[...] = y.reshape(64, T).T

    full = lambda w: pl.BlockSpec(w.shape, lambda i: (0,) * w.ndim)
    return pl.pallas_call(
        body,
        grid=(nt,),
        in_specs=[pl.BlockSpec((T, 64), lambda i: (i, 0)),
                  pl.BlockSpec((T, 64), lambda i: (i, 0)),
                  pl.BlockSpec((T, 16), lambda i: (i, 0)),
                  pl.BlockSpec((T, 16), lambda i: (i, 0))]
                 + [full(w) for w in wflat],
        out_specs=pl.BlockSpec((T, 64), lambda i: (i, 0)),
        out_shape=jax.ShapeDtypeStruct((N, 64), jnp.float32),
    )(hperm, sums, c0, c1, *wflat)


_SC_CH = 128


def _sc_gather(hperm, s2, r2, n_nodes):
    NROW = s2.shape[0]
    E = NROW * _SC_CH
    NW = 32
    NB = 2
    NIT = NROW // (NW * NB)
    assert NIT % 2 == 0 and NIT >= 4
    BE = NB * _SC_CH
    NS = 16
    NHPF = ((n_nodes + 1 + NS - 1) // NS) * NS
    RPT = NHPF // NS
    zeros16 = jnp.zeros((_SC_CH, 16), jnp.float32)
    ones16 = jnp.ones((_SC_CH, 16), jnp.float32)
    mesh = plsc.VectorSubcoreMesh(core_axis_name="c", subcore_axis_name="s",
                                  num_cores=2, num_subcores=16)

    @functools.partial(
        pl.kernel,
        out_type=(jax.ShapeDtypeStruct((E, 64), jnp.float32),
                  jax.ShapeDtypeStruct((E, 64), jnp.float32),
                  jax.ShapeDtypeStruct((2 * NHPF, 16), jnp.float32)),
        mesh=mesh,
        compiler_params=pltpu.CompilerParams(use_tc_tiling_on_sc=False),
        scratch_types=[
            pltpu.VMEM((2, NB, _SC_CH), jnp.int32),
            pltpu.VMEM((2, NB, _SC_CH), jnp.int32),
            pltpu.VMEM((2, BE, 64), jnp.float32),
            pltpu.VMEM((2, BE, 64), jnp.float32),
            pltpu.VMEM((_SC_CH, 16), jnp.float32),
            pltpu.VMEM_SHARED((NHPF, 16), jnp.float32),
            pltpu.SemaphoreType.DMA((2,)),
            pltpu.SemaphoreType.DMA((2,)),
            pltpu.SemaphoreType.DMA((2,)),
            pltpu.SemaphoreType.DMA((2,)),
        ],
    )
    def k(hp, sh, rh, z16_h, one_h, so, ro, co,
          sidx, ridx, srows, rrows, obuf, cnts_sh, semi, semg, semw, semc):
        c = lax.axis_index("c")
        s = lax.axis_index("s")
        wid = s * 2 + c
        row_base = wid * NIT * NB
        row0 = s * RPT

        pltpu.sync_copy(z16_h, obuf)
        o = 0
        while o < RPT:
            n = min(_SC_CH, RPT - o)
            pltpu.sync_copy(obuf.at[pl.ds(0, n)],
                            cnts_sh.at[pl.ds(row0 + o, n)])
            o += n
        pltpu.sync_copy(one_h, obuf)
        plsc.subcore_barrier()

        def idx_load(i, p):
            rr = row_base + i * NB
            pltpu.async_copy(sh.at[pl.ds(rr, NB)], sidx.at[p], semi.at[p])
            pltpu.async_copy(rh.at[pl.ds(rr, NB)], ridx.at[p], semi.at[p])

        def idx_wait(i, p):
            rr = row_base + i * NB
            pltpu.make_async_copy(sh.at[pl.ds(rr, NB)], sidx.at[p],
                                  semi.at[p]).wait()
            pltpu.make_async_copy(rh.at[pl.ds(rr, NB)], ridx.at[p],
                                  semi.at[p]).wait()

        def g_fire(p):
            for r in range(NB):
                pltpu.async_copy(hp.at[sidx.at[p, r]],
                                 srows.at[p, pl.ds(r * _SC_CH, _SC_CH)],
                                 semg.at[p])
                pltpu.async_copy(hp.at[ridx.at[p, r]],
                                 rrows.at[p, pl.ds(r * _SC_CH, _SC_CH)],
                                 semg.at[p])

        def g_wait(p):
            for r in range(NB):
                pltpu.make_async_copy(hp.at[sidx.at[p, r]],
                                      srows.at[p, pl.ds(r * _SC_CH, _SC_CH)],
                                      semg.at[p]).wait()
                pltpu.make_async_copy(hp.at[ridx.at[p, r]],
                                      rrows.at[p, pl.ds(r * _SC_CH, _SC_CH)],
                                      semg.at[p]).wait()

        def c_fire(p):
            for r in range(NB):
                pltpu.async_copy(obuf, cnts_sh.at[sidx.at[p, r]],
                                 semc.at[p], add=True)

        def c_drain(p):
            for r in range(NB):
                pltpu.make_async_copy(obuf, cnts_sh.at[sidx.at[p, r]],
                                      semc.at[p]).wait()

        def w_fire(i, p):
            off = (row_base + i * NB) * _SC_CH
            pltpu.async_copy(srows.at[p], so.at[pl.ds(off, BE)], semw.at[p])
            pltpu.async_copy(rrows.at[p], ro.at[pl.ds(off, BE)], semw.at[p])

        def w_drain(i, p):
            off = (row_base + i * NB) * _SC_CH
            pltpu.make_async_copy(srows.at[p], so.at[pl.ds(off, BE)],
                                  semw.at[p]).wait()
            pltpu.make_async_copy(rrows.at[p], ro.at[pl.ds(off, BE)],
                                  semw.at[p]).wait()

        idx_load(0, 0)
        idx_wait(0, 0)
        g_fire(0)
        c_fire(0)
        idx_load(1, 1)

        def step(i, p):
            g_wait(p)
            w_fire(i, p)

            @pl.when(i + 1 < NIT)
            def _():
                idx_wait(i + 1, 1 - p)

                @pl.when(i + 2 < NIT)
                def _():
                    c_drain(p)
                    idx_load(i + 2, p)

                @pl.when(i >= 1)
                def _():
                    w_drain(i - 1, 1 - p)

                g_fire(1 - p)
                c_fire(1 - p)

        def body(io, carry):
            step(io * 2, 0)
            step(io * 2 + 1, 1)
            return carry

        lax.fori_loop(0, NIT // 2, body, 0)
        c_drain(0)
        c_drain(1)
        w_drain(NIT - 2, 0)
        w_drain(NIT - 1, 1)
        plsc.subcore_barrier()

        out0 = c * NHPF + row0
        o = 0
        while o < RPT:
            n = min(_SC_CH, RPT - o)
            pltpu.sync_copy(cnts_sh.at[pl.ds(row0 + o, n)],
                            obuf.at[pl.ds(0, n)])
            pltpu.sync_copy(obuf.at[pl.ds(0, n)],
                            co.at[pl.ds(out0 + o, n)])
            o += n

    return k(hperm, s2, r2, zeros16, ones16), NHPF


def _sc_scatter(m, s2, n_nodes):
    NROW = s2.shape[0]
    NHALF = n_nodes // 2
    NS = 16
    NHP = ((NHALF + 1 + NS - 1) // NS) * NS
    NIT = NROW // NS
    assert NIT % 2 == 0
    RPT = NHP // NS
    zeros64 = jnp.zeros((_SC_CH, 64), jnp.float32)
    mesh = plsc.VectorSubcoreMesh(core_axis_name="c", subcore_axis_name="s",
                                  num_cores=2, num_subcores=16)

    @functools.partial(
        pl.kernel,
        out_type=jax.ShapeDtypeStruct((2 * NHP, 64), jnp.float32),
        mesh=mesh,
        compiler_params=pltpu.CompilerParams(use_tc_tiling_on_sc=False),
        scratch_types=[
            pltpu.VMEM((2, _SC_CH), jnp.int32),
            pltpu.VMEM((2, _SC_CH), jnp.int32),
            pltpu.VMEM((2, _SC_CH, 64), jnp.float32),
            pltpu.VMEM_SHARED((NHP, 64), jnp.float32),
            pltpu.SemaphoreType.DMA((2,)),
            pltpu.SemaphoreType.DMA((2,)),
        ],
    )
    def k(m_h, s_h, z64_h, sums_o, sidx, idxb, mbuf, sums_sh, seml, semsc):
        c = lax.axis_index("c")
        s = lax.axis_index("s")
        base = c * NHALF
        row0 = s * RPT

        pltpu.sync_copy(z64_h, mbuf.at[0])
        o = 0
        while o < RPT:
            n = min(_SC_CH, RPT - o)
            pltpu.sync_copy(mbuf.at[0, pl.ds(0, n)],
                            sums_sh.at[pl.ds(row0 + o, n)])
            o += n
        plsc.subcore_barrier()

        def m_load(i, p):
            r0 = s * NIT + i
            pltpu.async_copy(s_h.at[r0], sidx.at[p], seml.at[p])
            pltpu.async_copy(m_h.at[pl.ds(r0 * _SC_CH, _SC_CH)], mbuf.at[p],
                             seml.at[p])

        def m_wait(i, p):
            r0 = s * NIT + i
            pltpu.make_async_copy(s_h.at[r0], sidx.at[p], seml.at[p]).wait()
            pltpu.make_async_copy(m_h.at[pl.ds(r0 * _SC_CH, _SC_CH)],
                                  mbuf.at[p], seml.at[p]).wait()

        def sc_drain(p):
            pltpu.make_async_copy(mbuf.at[p], sums_sh.at[idxb.at[p]],
                                  semsc.at[p]).wait()

        m_load(0, 0)

        def step(i, p):
            m_wait(i, p)
            for v in range(_SC_CH // 16):
                sl = sidx[p, pl.ds(v * 16, 16)]
                loc = sl - base
                ok = (loc >= 0) & (loc < NHALF)
                idxb[p, pl.ds(v * 16, 16)] = jnp.where(ok, loc, NHALF)
            pltpu.async_copy(mbuf.at[p], sums_sh.at[idxb.at[p]],
                             semsc.at[p], add=True)

            @pl.when(i >= 1)
            def _():
                sc_drain(1 - p)

            @pl.when(i + 1 < NIT)
            def _():
                m_load(i + 1, 1 - p)

        def body(io, carry):
            step(io * 2, 0)
            step(io * 2 + 1, 1)
            return carry

        lax.fori_loop(0, NIT // 2, body, 0)
        sc_drain(1)
        plsc.subcore_barrier()

        out0 = c * NHP + row0
        o = 0
        while o < RPT:
            n = min(_SC_CH, RPT - o)
            pltpu.sync_copy(sums_sh.at[pl.ds(row0 + o, n)],
                            mbuf.at[0, pl.ds(0, n)])
            pltpu.sync_copy(mbuf.at[0, pl.ds(0, n)],
                            sums_o.at[pl.ds(out0 + o, n)])
            o += n

    sums2 = k(m, s2, zeros64)
    return jnp.concatenate([sums2[:NHALF], sums2[NHP:NHP + NHALF]], axis=0)


def kernel(h, edge_index, edge_attr, params):
    N, F = h.shape[0], h.shape[1]
    E = edge_index.shape[1]
    EC = edge_attr.shape[1]
    assert F == 8 and EC == 4 and N % 2 == 0

    phie = [_prep_layer(p, pc) for p, pc in zip(params["phi_e"], [(8, 4), (8,)])]
    phih = [_prep_layer(p, pc) for p, pc in zip(params["phi_h"], [(8, 8), (8,)])]
    we = _flatten_layers(phie)
    wh = _flatten_layers(phih)

    hperm = jnp.transpose(h, (0, 2, 1)).reshape(N, 64)
    aperm = jnp.transpose(edge_attr, (0, 2, 1)).reshape(E, 32)

    EPAD = ((E + 32 * _SC_CH - 1) // (32 * _SC_CH)) * (32 * _SC_CH)
    senders = edge_index[0]
    NR = EPAD // _SC_CH
    s2 = jnp.pad(senders, (0, EPAD - E)).reshape(NR, _SC_CH)
    r2 = jnp.pad(edge_index[1], (0, EPAD - E)).reshape(NR, _SC_CH)
    s2s = jnp.pad(senders, (0, EPAD - E),
                  constant_values=N).reshape(NR, _SC_CH)
    aperm = jnp.pad(aperm, ((0, EPAD - E), (0, 0)))

    hperm_p8 = jnp.pad(hperm, ((0, 8), (0, 0)))
    (srows, rrows, cnts2), NHPF = _sc_gather(hperm_p8, s2s, r2, N)

    m_e = _phie_call(srows, rrows, aperm, we, T=4096)

    sums = _sc_scatter(m_e, s2s, N)

    TN = 1024
    NPAD = ((N + TN - 1) // TN) * TN
    hperm_p = jnp.pad(hperm, ((0, NPAD - N), (0, 0)))
    sums_p = jnp.pad(sums, ((0, NPAD - N), (0, 0)))
    c0_p = jnp.pad(cnts2[:N], ((0, NPAD - N), (0, 0)))
    c1_p = jnp.pad(cnts2[NHPF:NHPF + N], ((0, NPAD - N), (0, 0)))
    y = _phih_call(hperm_p, sums_p, c0_p, c1_p, wh, T=TN)

    return jnp.transpose(y[:N].reshape(N, 8, 8), (0, 2, 1))
